# baseline (speedup 1.0000x reference)
import jax
import jax.numpy as jnp
from jax.experimental import pallas as pl
from jax.experimental.pallas import tpu as pltpu


def kernel(x, offsets):
    tokens, d = x.shape
    nseg = offsets.shape[0] - 1

    def _pool(offs_ref, x_ref, out_ref, sem):
        copies = []
        for i in range(nseg):
            row = offs_ref[i + 1] - 1
            copies.append(
                pltpu.make_async_copy(
                    x_ref.at[pl.ds(row, 1)], out_ref.at[pl.ds(i, 1)], sem
                )
            )
        for c in copies:
            c.start()
        for c in copies:
            c.wait()

    grid_spec = pltpu.PrefetchScalarGridSpec(
        num_scalar_prefetch=1,
        grid=(1,),
        in_specs=[pl.BlockSpec(memory_space=pl.ANY)],
        out_specs=pl.BlockSpec((nseg, d), lambda i, offs: (0, 0)),
        scratch_shapes=[pltpu.SemaphoreType.DMA],
    )

    return pl.pallas_call(
        _pool,
        grid_spec=grid_spec,
        out_shape=jax.ShapeDtypeStruct((nseg, d), x.dtype),
    )(offsets.astype(jnp.int32), x)
